# Initial kernel scaffold; baseline (speedup 1.0000x reference)
import jax
import jax.numpy as jnp
from jax.experimental import pallas as pl


def kernel(x, edge_index, W1, b1, W2, b2):
    raise NotImplementedError("write your pallas kernel here")


# speedup vs baseline: 11.8320x; 11.8320x over previous
import functools

import jax
import jax.numpy as jnp
from jax import lax
from jax.experimental import pallas as pl
from jax.experimental.pallas import tpu as pltpu
from jax.experimental.pallas import tpu_sc as plsc

_N = 10000
_E = 320000
_D_IN = 128
_HID = 128
_C = 64

_NC = 2
_NS = 16
_NW = _NC * _NS
_L = 16

_N_PAD = 10240
_SR = _N_PAD // _NS
_CHUNK = 128
_CH = 79
_E_PAD = _NW * _CH * _CHUNK
_RB = 1024
_DEG_W = 128
_C_PAD = 128

_mesh = plsc.VectorSubcoreMesh(core_axis_name="c", subcore_axis_name="s")


def _sc_degree(dst_hbm):
    @functools.partial(
        pl.kernel,
        out_type=jax.ShapeDtypeStruct((_NC * _N_PAD, _DEG_W), jnp.float32),
        mesh=_mesh,
        scratch_types=[
            pltpu.VMEM((_CH, _CHUNK), jnp.int32),
            pltpu.VMEM((_CHUNK, _DEG_W), jnp.float32),
            pltpu.VMEM((_CHUNK, _DEG_W), jnp.float32),
            pltpu.VMEM_SHARED((_N_PAD, _DEG_W), jnp.float32),
            pltpu.SemaphoreType.DMA,
        ],
    )
    def k(dst_idx, out_hbm, dst_v, ones_v, zb_v, acc, sem):
        c = lax.axis_index("c")
        s = lax.axis_index("s")
        w = c * _NS + s

        @pl.loop(0, _CHUNK)
        def _(i):
            @pl.loop(0, _DEG_W, step=_L)
            def _(kk):
                ones_v[i, pl.ds(kk, _L)] = jnp.full((_L,), 1.0, jnp.float32)
                zb_v[i, pl.ds(kk, _L)] = jnp.zeros((_L,), jnp.float32)

        @pl.loop(0, _SR // _CHUNK)
        def _(t):
            pltpu.sync_copy(zb_v, acc.at[pl.ds(s * _SR + t * _CHUNK, _CHUNK)])

        pltpu.sync_copy(dst_idx.at[w], dst_v)
        plsc.subcore_barrier()

        @pl.loop(0, _CH)
        def _(j):
            pltpu.sync_copy(ones_v, acc.at[dst_v.at[j]], add=True)

        plsc.subcore_barrier()
        pltpu.sync_copy(acc.at[pl.ds(s * _SR, _SR)],
                        out_hbm.at[pl.ds(c * _N_PAD + s * _SR, _SR)])

    return k(dst_hbm)


def _sc_aggregate(g_hbm, src_hbm, dst_hbm, d):
    @functools.partial(
        pl.kernel,
        out_type=jax.ShapeDtypeStruct((_NC * _N_PAD, d), jnp.float32),
        mesh=_mesh,
        scratch_types=[
            pltpu.VMEM((_CH, _CHUNK), jnp.int32),
            pltpu.VMEM((_CH, _CHUNK), jnp.int32),
            pltpu.VMEM((_CHUNK, d), jnp.float32),
            pltpu.VMEM_SHARED((_N_PAD, d), jnp.float32),
            pltpu.SemaphoreType.DMA,
        ],
    )
    def k(g, src_idx, dst_idx, out_hbm, src_v, dst_v, rows_v, acc, sem):
        c = lax.axis_index("c")
        s = lax.axis_index("s")
        w = c * _NS + s

        @pl.loop(0, _CHUNK)
        def _(i):
            @pl.loop(0, d, step=_L)
            def _(kk):
                rows_v[i, pl.ds(kk, _L)] = jnp.zeros((_L,), jnp.float32)

        @pl.loop(0, _SR // _CHUNK)
        def _(t):
            pltpu.sync_copy(rows_v, acc.at[pl.ds(s * _SR + t * _CHUNK, _CHUNK)])

        pltpu.sync_copy(src_idx.at[w], src_v)
        pltpu.sync_copy(dst_idx.at[w], dst_v)
        plsc.subcore_barrier()

        @pl.loop(0, _CH)
        def _(j):
            pltpu.async_copy(g.at[src_v.at[j]], rows_v, sem).wait()
            pltpu.sync_copy(rows_v, acc.at[dst_v.at[j]], add=True)

        plsc.subcore_barrier()
        pltpu.sync_copy(acc.at[pl.ds(s * _SR, _SR)],
                        out_hbm.at[pl.ds(c * _N_PAD + s * _SR, _SR)])

    return k(g_hbm, src_hbm, dst_hbm)


def _mm_body(x_ref, w_ref, o_ref):
    o_ref[...] = jnp.dot(x_ref[...], w_ref[...],
                         preferred_element_type=jnp.float32,
                         precision=lax.Precision.HIGHEST)


def _tc_matmul(x_pad, W):
    d_in, d_out = W.shape
    return pl.pallas_call(
        _mm_body,
        grid=(_N_PAD // _RB,),
        in_specs=[pl.BlockSpec((_RB, d_in), lambda i: (i, 0)),
                  pl.BlockSpec((d_in, d_out), lambda i: (0, 0))],
        out_specs=pl.BlockSpec((_RB, d_out), lambda i: (i, 0)),
        out_shape=jax.ShapeDtypeStruct((_N_PAD, d_out), jnp.float32),
    )(x_pad, W)


def _dis_scale_body(d0_ref, d1_ref, h_ref, dis_ref, g_ref):
    dsum = d0_ref[0, :, 0:1] + d1_ref[0, :, 0:1] + 1.0
    dis = lax.rsqrt(dsum)
    dis_ref[...] = dis
    g_ref[...] = h_ref[...] * dis


def _tc_dis_scale(deg2, h1):
    return pl.pallas_call(
        _dis_scale_body,
        grid=(_N_PAD // _RB,),
        in_specs=[pl.BlockSpec((1, _RB, _DEG_W), lambda i: (0, i, 0)),
                  pl.BlockSpec((1, _RB, _DEG_W), lambda i: (1, i, 0)),
                  pl.BlockSpec((_RB, _HID), lambda i: (i, 0))],
        out_specs=[pl.BlockSpec((_RB, 1), lambda i: (i, 0)),
                   pl.BlockSpec((_RB, _HID), lambda i: (i, 0))],
        out_shape=[jax.ShapeDtypeStruct((_N_PAD, 1), jnp.float32),
                   jax.ShapeDtypeStruct((_N_PAD, _HID), jnp.float32)],
    )(deg2, deg2, h1)


def _layer1_body(p0_ref, p1_ref, g1_ref, dis_ref, b1_ref, w2_ref, g2_ref):
    ssum = p0_ref[0] + p1_ref[0] + g1_ref[...]
    z = jnp.maximum(ssum * dis_ref[...] + b1_ref[...], 0.0)
    h2 = jnp.dot(z, w2_ref[...], preferred_element_type=jnp.float32,
                 precision=lax.Precision.HIGHEST)
    g2_ref[...] = h2 * dis_ref[...]


def _tc_layer1_combine(p, g1, dis, b1, W2):
    return pl.pallas_call(
        _layer1_body,
        grid=(_N_PAD // _RB,),
        in_specs=[pl.BlockSpec((1, _RB, _HID), lambda i: (0, i, 0)),
                  pl.BlockSpec((1, _RB, _HID), lambda i: (1, i, 0)),
                  pl.BlockSpec((_RB, _HID), lambda i: (i, 0)),
                  pl.BlockSpec((_RB, 1), lambda i: (i, 0)),
                  pl.BlockSpec((1, _HID), lambda i: (0, 0)),
                  pl.BlockSpec((_HID, _C_PAD), lambda i: (0, 0))],
        out_specs=pl.BlockSpec((_RB, _C_PAD), lambda i: (i, 0)),
        out_shape=jax.ShapeDtypeStruct((_N_PAD, _C_PAD), jnp.float32),
    )(p, p, g1, dis, b1, W2)


def _layer2_body(q0_ref, q1_ref, g2_ref, dis_ref, b2_ref, o_ref):
    ssum = q0_ref[0] + q1_ref[0] + g2_ref[...]
    o_ref[...] = ssum * dis_ref[...] + b2_ref[...]


def _tc_layer2_combine(q, g2, dis, b2):
    return pl.pallas_call(
        _layer2_body,
        grid=(_N_PAD // _RB,),
        in_specs=[pl.BlockSpec((1, _RB, _C_PAD), lambda i: (0, i, 0)),
                  pl.BlockSpec((1, _RB, _C_PAD), lambda i: (1, i, 0)),
                  pl.BlockSpec((_RB, _C_PAD), lambda i: (i, 0)),
                  pl.BlockSpec((_RB, 1), lambda i: (i, 0)),
                  pl.BlockSpec((1, _C_PAD), lambda i: (0, 0))],
        out_specs=pl.BlockSpec((_RB, _C_PAD), lambda i: (i, 0)),
        out_shape=jax.ShapeDtypeStruct((_N_PAD, _C_PAD), jnp.float32),
    )(q, q, g2, dis, b2)


def kernel(x, edge_index, W1, b1, W2, b2):
    src = edge_index[0]
    dst = edge_index[1]
    npad = _E_PAD - _E
    pad_idx = jnp.full((npad,), _N, jnp.int32)
    src_p = jnp.concatenate([src, pad_idx]).reshape(_NW, _CH, _CHUNK)
    dst_p = jnp.concatenate([dst, pad_idx]).reshape(_NW, _CH, _CHUNK)
    x_pad = jnp.zeros((_N_PAD, _D_IN), jnp.float32).at[:_N].set(x)

    W2p = jnp.zeros((_HID, _C_PAD), jnp.float32).at[:, :_C].set(W2)
    b2p = jnp.zeros((1, _C_PAD), jnp.float32).at[:, :_C].set(b2)

    deg2 = _sc_degree(dst_p).reshape(_NC, _N_PAD, _DEG_W)
    h1 = _tc_matmul(x_pad, W1)
    dis, g1 = _tc_dis_scale(deg2, h1)

    p = _sc_aggregate(g1, src_p, dst_p, _HID).reshape(_NC, _N_PAD, _HID)
    g2 = _tc_layer1_combine(p, g1, dis, b1.reshape(1, _HID), W2p)

    q = _sc_aggregate(g2, src_p, dst_p, _C_PAD).reshape(_NC, _N_PAD, _C_PAD)
    out = _tc_layer2_combine(q, g2, dis, b2p)
    return out[:_N, :_C]


# speedup vs baseline: 11.9027x; 1.0060x over previous
import dataclasses
import functools

import jax
import jax.numpy as jnp
from jax import lax
from jax.experimental import pallas as pl
from jax.experimental.pallas import tpu as pltpu
from jax.experimental.pallas import tpu_sc as plsc

_N = 10000
_E = 320000
_D_IN = 128
_HID = 128
_C = 64

_NC = 2
_NS = 16
_NW = _NC * _NS
_L = 16

_N_PAD = 10240
_SR = _N_PAD // _NS
_CHUNK = 128
_CH = 80
_NBUF = 4
_E_PAD = _NW * _CH * _CHUNK
_RB = 1024
_DEG_W = 128
_C_PAD = 128

_mesh = plsc.VectorSubcoreMesh(core_axis_name="c", subcore_axis_name="s")

_sc_params = pltpu.CompilerParams()
if "needs_layout_passes" in pltpu.CompilerParams.__dataclass_fields__:
    _sc_params = dataclasses.replace(_sc_params, needs_layout_passes=False)
_sc_params = dataclasses.replace(_sc_params, internal_scratch_in_bytes=0)
_sc_linear_params = pltpu.CompilerParams()
if "use_tc_tiling_on_sc" in pltpu.CompilerParams.__dataclass_fields__:
    _sc_linear_params = dataclasses.replace(
        _sc_linear_params, use_tc_tiling_on_sc=False)


def _sc_degree(dst_hbm):
    @functools.partial(
        pl.kernel,
        out_type=jax.ShapeDtypeStruct((_NW, _N_PAD), jnp.float32),
        mesh=_mesh,
        compiler_params=_sc_params,
        scratch_types=[
            pltpu.VMEM((_CH, _CHUNK), jnp.int32),
            pltpu.VMEM((_N_PAD,), jnp.float32),
        ],
    )
    def k(dst_idx, out_hbm, dst_v, hist):
        c = lax.axis_index("c")
        s = lax.axis_index("s")
        w = c * _NS + s

        @pl.loop(0, _N_PAD, step=_L)
        def _(i):
            hist[pl.ds(i, _L)] = jnp.zeros((_L,), jnp.float32)

        pltpu.sync_copy(dst_idx.at[w], dst_v)
        ones = jnp.full((_L,), 1.0, jnp.float32)

        @pl.loop(0, _CH)
        def _(j):
            @pl.loop(0, _CHUNK, step=_L)
            def _(kk):
                idx = dst_v[j, pl.ds(kk, _L)]
                plsc.addupdate_scatter(hist, [idx], ones)

        pltpu.sync_copy(hist, out_hbm.at[w])

    return k(dst_hbm)


def _sc_aggregate(g_hbm, src_hbm, dst_hbm, d, linear=False, pipelined=True):
    @functools.partial(
        pl.kernel,
        out_type=jax.ShapeDtypeStruct((_NC * _N_PAD, d), jnp.float32),
        mesh=_mesh,
        compiler_params=_sc_linear_params if linear else None,
        scratch_types=[
            pltpu.VMEM((_CH, _CHUNK), jnp.int32),
            pltpu.VMEM((_CH, _CHUNK), jnp.int32),
            pltpu.VMEM((_CHUNK, d), jnp.float32),
            pltpu.VMEM((_CHUNK, d), jnp.float32),
            pltpu.VMEM((_CHUNK, d), jnp.float32),
            pltpu.VMEM((_CHUNK, d), jnp.float32),
            pltpu.VMEM_SHARED((_N_PAD, d), jnp.float32),
            pltpu.SemaphoreType.DMA,
            pltpu.SemaphoreType.DMA,
            pltpu.SemaphoreType.DMA,
            pltpu.SemaphoreType.DMA,
        ],
    )
    def k(g, src_idx, dst_idx, out_hbm, src_v, dst_v, r0, r1, r2, r3, acc,
          gsem, gsem2, gsem3, gsem4):
        rows = (r0, r1, r2, r3)
        c = lax.axis_index("c")
        s = lax.axis_index("s")
        w = c * _NS + s

        @pl.loop(0, _CHUNK)
        def _(i):
            @pl.loop(0, d, step=_L)
            def _(kk):
                r0[i, pl.ds(kk, _L)] = jnp.zeros((_L,), jnp.float32)

        @pl.loop(0, _SR // _CHUNK)
        def _(t):
            pltpu.sync_copy(r0, acc.at[pl.ds(s * _SR + t * _CHUNK, _CHUNK)])

        pltpu.sync_copy(src_idx.at[w], src_v)
        pltpu.sync_copy(dst_idx.at[w], dst_v)
        plsc.subcore_barrier()

        if pipelined:
            gsems = (gsem, gsem2, gsem3, gsem4)
            ngroups = _CH // _NBUF

            @pl.loop(0, ngroups)
            def _(t):
                base = t * _NBUF
                for b in range(_NBUF):
                    pltpu.async_copy(g.at[src_v.at[base + b]], rows[b],
                                     gsems[b])
                for b in range(_NBUF):
                    pltpu.make_async_copy(g.at[src_v.at[0]], rows[b],
                                          gsems[b]).wait()
                    pltpu.sync_copy(rows[b],
                                    acc.at[dst_v.at[base + b]], add=True)
        else:
            @pl.loop(0, _CH)
            def _(j):
                pltpu.async_copy(g.at[src_v.at[j]], r0, gsem).wait()
                pltpu.sync_copy(r0, acc.at[dst_v.at[j]], add=True)

        plsc.subcore_barrier()
        pltpu.sync_copy(acc.at[pl.ds(s * _SR, _SR)],
                        out_hbm.at[pl.ds(c * _N_PAD + s * _SR, _SR)])

    return k(g_hbm, src_hbm, dst_hbm)


def _mm_body(x_ref, w_ref, o_ref):
    o_ref[...] = jnp.dot(x_ref[...], w_ref[...],
                         preferred_element_type=jnp.float32,
                         precision=lax.Precision.HIGHEST)


def _tc_matmul(x_pad, W):
    d_in, d_out = W.shape
    return pl.pallas_call(
        _mm_body,
        grid=(_N_PAD // _RB,),
        in_specs=[pl.BlockSpec((_RB, d_in), lambda i: (i, 0)),
                  pl.BlockSpec((d_in, d_out), lambda i: (0, 0))],
        out_specs=pl.BlockSpec((_RB, d_out), lambda i: (i, 0)),
        out_shape=jax.ShapeDtypeStruct((_N_PAD, d_out), jnp.float32),
    )(x_pad, W)


def _dis_scale_body(dt_ref, h_ref, dis_ref, g_ref):
    dsum = jnp.sum(dt_ref[...], axis=1, keepdims=True) + 1.0
    dis = lax.rsqrt(dsum)
    dis_ref[...] = dis
    g_ref[...] = h_ref[...] * dis


def _tc_dis_scale(deg_t, h1):
    return pl.pallas_call(
        _dis_scale_body,
        grid=(_N_PAD // _RB,),
        in_specs=[pl.BlockSpec((_RB, _NW), lambda i: (i, 0)),
                  pl.BlockSpec((_RB, _HID), lambda i: (i, 0))],
        out_specs=[pl.BlockSpec((_RB, 1), lambda i: (i, 0)),
                   pl.BlockSpec((_RB, _HID), lambda i: (i, 0))],
        out_shape=[jax.ShapeDtypeStruct((_N_PAD, 1), jnp.float32),
                   jax.ShapeDtypeStruct((_N_PAD, _HID), jnp.float32)],
    )(deg_t, h1)


def _layer1_body(p0_ref, p1_ref, g1_ref, dis_ref, b1_ref, w2_ref, g2_ref):
    ssum = p0_ref[0] + p1_ref[0] + g1_ref[...]
    z = jnp.maximum(ssum * dis_ref[...] + b1_ref[...], 0.0)
    h2 = jnp.dot(z, w2_ref[...], preferred_element_type=jnp.float32,
                 precision=lax.Precision.HIGHEST)
    g2_ref[...] = h2 * dis_ref[...]


def _tc_layer1_combine(p, g1, dis, b1, W2):
    return pl.pallas_call(
        _layer1_body,
        grid=(_N_PAD // _RB,),
        in_specs=[pl.BlockSpec((1, _RB, _HID), lambda i: (0, i, 0)),
                  pl.BlockSpec((1, _RB, _HID), lambda i: (1, i, 0)),
                  pl.BlockSpec((_RB, _HID), lambda i: (i, 0)),
                  pl.BlockSpec((_RB, 1), lambda i: (i, 0)),
                  pl.BlockSpec((1, _HID), lambda i: (0, 0)),
                  pl.BlockSpec((_HID, _C), lambda i: (0, 0))],
        out_specs=pl.BlockSpec((_RB, _C), lambda i: (i, 0)),
        out_shape=jax.ShapeDtypeStruct((_N_PAD, _C), jnp.float32),
    )(p, p, g1, dis, b1, W2)


def _layer2_body(q0_ref, q1_ref, g2_ref, dis_ref, b2_ref, o_ref):
    ssum = q0_ref[0] + q1_ref[0] + g2_ref[...]
    o_ref[...] = ssum * dis_ref[...] + b2_ref[...]


def _tc_layer2_combine(q, g2, dis, b2):
    return pl.pallas_call(
        _layer2_body,
        grid=(_N_PAD // _RB,),
        in_specs=[pl.BlockSpec((1, _RB, _C), lambda i: (0, i, 0)),
                  pl.BlockSpec((1, _RB, _C), lambda i: (1, i, 0)),
                  pl.BlockSpec((_RB, _C), lambda i: (i, 0)),
                  pl.BlockSpec((_RB, 1), lambda i: (i, 0)),
                  pl.BlockSpec((1, _C), lambda i: (0, 0))],
        out_specs=pl.BlockSpec((_RB, _C), lambda i: (i, 0)),
        out_shape=jax.ShapeDtypeStruct((_N_PAD, _C), jnp.float32),
    )(q, q, g2, dis, b2)


def kernel(x, edge_index, W1, b1, W2, b2):
    src = edge_index[0]
    dst = edge_index[1]
    npad = _E_PAD - _E
    pad_idx = jnp.full((npad,), _N, jnp.int32)
    src_p = jnp.concatenate([src, pad_idx]).reshape(_NW, _CH, _CHUNK)
    dst_p = jnp.concatenate([dst, pad_idx]).reshape(_NW, _CH, _CHUNK)
    x_pad = jnp.zeros((_N_PAD, _D_IN), jnp.float32).at[:_N].set(x)

    deg_t = _sc_degree(dst_p).T
    h1 = _tc_matmul(x_pad, W1)
    dis, g1 = _tc_dis_scale(deg_t, h1)

    p = _sc_aggregate(g1, src_p, dst_p, _HID,
                      pipelined=False).reshape(_NC, _N_PAD, _HID)
    g2 = _tc_layer1_combine(p, g1, dis, b1.reshape(1, _HID), W2)

    q = _sc_aggregate(g2, src_p, dst_p, _C, linear=True,
                      pipelined=False).reshape(_NC, _N_PAD, _C)
    out = _tc_layer2_combine(q, g2, dis, b2.reshape(1, _C))
    return out[:_N]
